# baseline (speedup 1.0000x reference)
import jax
import jax.numpy as jnp
from jax import lax
from jax.experimental import pallas as pl
from jax.experimental.pallas import tpu as pltpu
from jax.experimental.pallas import tpu_sc as plsc

SEP = 103
D = 768
SEQ = 512
B = 32
L = 16
NC, NS = 2, 16
RPW = 8
PW = 64
CS = 16
NSLOT = 4
DCH = D // L


def _body(ids_hbm, table_hbm, seg_hbm, pos_hbm, out_hbm,
          ids_v, idxw_v, t_v, seg_v, pos_win, tok0, tok1, tok2, tok3,
          perm_v, sg0, sg1, sg2, sg3, so0, so1, so2, so3, sem_misc):
    c = lax.axis_index("c")
    s = lax.axis_index("s")
    w = s * NC + c
    bg = w // RPW
    pb = w % RPW
    p0 = pb * PW

    pltpu.sync_copy(ids_hbm.at[pl.ds(bg * RPW, RPW)], ids_v)
    for rr in range(RPW):
        for q in range(PW // L):
            idxw_v[rr, pl.ds(q * L, L)] = ids_v[rr, pl.ds(p0 + q * L, L)]
    pltpu.sync_copy(seg_hbm, seg_v)
    pltpu.async_copy(pos_hbm.at[pl.ds(p0, PW)], pos_win, sem_misc)

    iota = lax.iota(jnp.int32, L)
    first_chunk = pb * (PW // L)

    for r in range(RPW):
        def pre_body(i, acc):
            tok = ids_v[r, pl.ds(i * L, L)]
            return acc | jnp.where(tok == SEP, 1, 0).astype(jnp.int32)

        acc = lax.fori_loop(0, first_chunk, pre_body,
                            jnp.zeros((L,), jnp.int32))
        carry = acc
        for sh in (1, 2, 4, 8):
            perm_v[...] = carry
            carry = carry | plsc.load_gather(perm_v, [iota ^ sh])

        for q in range(PW // L):
            tok = ids_v[r, pl.ds((first_chunk + q) * L, L)]
            p = jnp.where(tok == SEP, 1, 0).astype(jnp.int32)
            for sh in (1, 2, 4, 8):
                perm_v[...] = p
                p = p | plsc.load_gather(perm_v,
                                         [jnp.maximum(iota - sh, 0)])
            perm_v[...] = p
            excl = plsc.load_gather(perm_v, [jnp.maximum(iota - 1, 0)])
            excl = jnp.where(iota == 0, 0, excl)
            t_v[r, pl.ds(q * L, L)] = (carry | excl).astype(jnp.float32)
            last = plsc.load_gather(perm_v,
                                    [jnp.full((L,), L - 1, jnp.int32)])
            carry = carry | last

    pltpu.make_async_copy(pos_hbm.at[pl.ds(p0, PW)], pos_win,
                          sem_misc).wait()

    def prefetch(r, o, tok_b, sem_g):
        pltpu.async_copy(
            table_hbm.at[idxw_v.at[r, pl.ds(o, CS)]], tok_b, sem_g)

    def compute(r, o, tok_b, sem_g, sem_o):
        b = bg * RPW + r
        pltpu.make_async_copy(
            table_hbm.at[idxw_v.at[r, pl.ds(o, CS)]], tok_b,
            sem_g).wait()

        tfirst = t_v[r, pl.ds(o, L)][0]
        tlast = t_v[r, pl.ds(o, L)][L - 1]
        uniform = tfirst == tlast

        @pl.when(uniform)
        def _():
            def dloop(j, _):
                dsl = pl.ds(j * L, L)
                seg0 = seg_v[0, dsl]
                segj = seg0 + tfirst * (seg_v[1, dsl] - seg0)
                for k in range(L):
                    plsc.addupdate(tok_b.at[k, dsl],
                                   pos_win[o + k, dsl] + segj)
                return 0
            lax.fori_loop(0, DCH, dloop, 0, unroll=2)

        @pl.when(jnp.logical_not(uniform))
        def _():
            def dloop(j, _):
                dsl = pl.ds(j * L, L)
                seg0 = seg_v[0, dsl]
                dseg = seg_v[1, dsl] - seg0
                tvec = t_v[r, pl.ds(o, L)]
                for k in range(L):
                    plsc.addupdate(tok_b.at[k, dsl],
                                   pos_win[o + k, dsl]
                                   + (seg0 + tvec[k] * dseg))
                return 0
            lax.fori_loop(0, DCH, dloop, 0)

        pltpu.async_copy(tok_b, out_hbm.at[b, pl.ds(p0 + o, CS)], sem_o)

    def wait_out(tok_b, sem_o):
        pltpu.make_async_copy(tok_b, out_hbm.at[0, pl.ds(0, CS)],
                              sem_o).wait()

    toks = (tok0, tok1, tok2, tok3)
    sgs = (sg0, sg1, sg2, sg3)
    sos = (so0, so1, so2, so3)

    for q in range(NSLOT):
        prefetch(0, q * CS, toks[q], sgs[q])

    def rowstep(r, _):
        compute(r, 0, toks[0], sgs[0], sos[0])

        @pl.when(r > 0)
        def _():
            wait_out(toks[3], sos[3])
            prefetch(r, 3 * CS, toks[3], sgs[3])

        for q in range(1, NSLOT):
            compute(r, q * CS, toks[q], sgs[q], sos[q])

            @pl.when(r < RPW - 1)
            def _():
                wait_out(toks[q - 1], sos[q - 1])
                prefetch(r + 1, (q - 1) * CS, toks[q - 1], sgs[q - 1])

        return 0

    lax.fori_loop(0, RPW, rowstep, 0)
    for q in range(NSLOT):
        wait_out(toks[q], sos[q])


@jax.jit
def kernel(token_ids, token_matrix, segment_matrix, pos_matrix):
    mesh = plsc.VectorSubcoreMesh(core_axis_name="c", subcore_axis_name="s",
                                  num_cores=NC, num_subcores=NS)
    run = pl.kernel(
        _body,
        out_type=jax.ShapeDtypeStruct((B, SEQ, D), jnp.float32),
        mesh=mesh,
        scratch_types=[
            pltpu.VMEM((RPW, SEQ), jnp.int32),
            pltpu.VMEM((RPW, PW), jnp.int32),
            pltpu.VMEM((RPW, PW), jnp.float32),
            pltpu.VMEM((2, D), jnp.float32),
            pltpu.VMEM((PW, D), jnp.float32),
            pltpu.VMEM((CS, D), jnp.float32),
            pltpu.VMEM((CS, D), jnp.float32),
            pltpu.VMEM((CS, D), jnp.float32),
            pltpu.VMEM((CS, D), jnp.float32),
            pltpu.VMEM((L,), jnp.int32),
            pltpu.SemaphoreType.DMA,
            pltpu.SemaphoreType.DMA,
            pltpu.SemaphoreType.DMA,
            pltpu.SemaphoreType.DMA,
            pltpu.SemaphoreType.DMA,
            pltpu.SemaphoreType.DMA,
            pltpu.SemaphoreType.DMA,
            pltpu.SemaphoreType.DMA,
            pltpu.SemaphoreType.DMA,
        ],
        compiler_params=pltpu.CompilerParams(needs_layout_passes=False),
    )
    return run(token_ids.astype(jnp.int32), token_matrix, segment_matrix,
               pos_matrix)


# speedup vs baseline: 1.5329x; 1.5329x over previous
import jax
import jax.numpy as jnp
from jax import lax
from jax.experimental import pallas as pl
from jax.experimental.pallas import tpu as pltpu
from jax.experimental.pallas import tpu_sc as plsc

SEP = 103
D = 768
SEQ = 512
B = 32
L = 16
NC, NS = 2, 16
RPW = 8
PW = 64
CS = 16
NSLOT = 4
DCH = D // L


def _body(ids_hbm, table_hbm, seg_hbm, pos_hbm, out_hbm,
          ids_v, idxw_v, t_v, seg_v, pos_win, tok0, tok1, tok2, tok3,
          perm_v, sg0, sg1, sg2, sg3, so0, so1, so2, so3, sem_misc):
    c = lax.axis_index("c")
    s = lax.axis_index("s")
    w = s * NC + c
    bg = w // RPW
    pb = w % RPW
    p0 = pb * PW

    pltpu.sync_copy(ids_hbm.at[pl.ds(bg * RPW, RPW)], ids_v)
    for rr in range(RPW):
        for q in range(PW // L):
            idxw_v[rr, pl.ds(q * L, L)] = ids_v[rr, pl.ds(p0 + q * L, L)]
    pltpu.sync_copy(seg_hbm, seg_v)
    pltpu.async_copy(pos_hbm.at[pl.ds(p0, PW)], pos_win, sem_misc)

    for q0 in range(NSLOT):
        pltpu.async_copy(
            table_hbm.at[idxw_v.at[0, pl.ds(q0 * CS, CS)]],
            (tok0, tok1, tok2, tok3)[q0],
            (sg0, sg1, sg2, sg3)[q0])

    iota = lax.iota(jnp.int32, L)
    first_chunk = pb * (PW // L)

    for r in range(RPW):
        def pre_body(i, acc):
            tok = ids_v[r, pl.ds(i * L, L)]
            return acc | jnp.where(tok == SEP, 1, 0).astype(jnp.int32)

        acc = lax.fori_loop(0, first_chunk, pre_body,
                            jnp.zeros((L,), jnp.int32))
        carry = acc
        for sh in (1, 2, 4, 8):
            perm_v[...] = carry
            carry = carry | plsc.load_gather(perm_v, [iota ^ sh])

        for q in range(PW // L):
            tok = ids_v[r, pl.ds((first_chunk + q) * L, L)]
            p = jnp.where(tok == SEP, 1, 0).astype(jnp.int32)
            for sh in (1, 2, 4, 8):
                perm_v[...] = p
                p = p | plsc.load_gather(perm_v,
                                         [jnp.maximum(iota - sh, 0)])
            perm_v[...] = p
            excl = plsc.load_gather(perm_v, [jnp.maximum(iota - 1, 0)])
            excl = jnp.where(iota == 0, 0, excl)
            t_v[r, pl.ds(q * L, L)] = (carry | excl).astype(jnp.float32)
            last = plsc.load_gather(perm_v,
                                    [jnp.full((L,), L - 1, jnp.int32)])
            carry = carry | last

    pltpu.make_async_copy(pos_hbm.at[pl.ds(p0, PW)], pos_win,
                          sem_misc).wait()

    def prefetch(r, o, tok_b, sem_g):
        pltpu.async_copy(
            table_hbm.at[idxw_v.at[r, pl.ds(o, CS)]], tok_b, sem_g)

    def compute(r, o, tok_b, sem_g, sem_o):
        b = bg * RPW + r
        pltpu.make_async_copy(
            table_hbm.at[idxw_v.at[r, pl.ds(o, CS)]], tok_b,
            sem_g).wait()

        tfirst = t_v[r, pl.ds(o, L)][0]
        tlast = t_v[r, pl.ds(o, L)][L - 1]
        uniform = tfirst == tlast

        @pl.when(uniform)
        def _():
            tsel = jnp.where(tfirst > 0.5, 1, 0)

            def dloop(j, _):
                dsl = pl.ds(j * L, L)
                segj = seg_v[tsel, dsl]
                for k in range(L):
                    plsc.addupdate(tok_b.at[k, dsl],
                                   pos_win[o + k, dsl] + segj)
                return 0
            lax.fori_loop(0, DCH, dloop, 0)

        @pl.when(jnp.logical_not(uniform))
        def _():
            def dloop(j, _):
                dsl = pl.ds(j * L, L)
                seg0 = seg_v[0, dsl]
                dseg = seg_v[1, dsl] - seg0
                tvec = t_v[r, pl.ds(o, L)]
                for k in range(L):
                    plsc.addupdate(tok_b.at[k, dsl],
                                   pos_win[o + k, dsl]
                                   + (seg0 + tvec[k] * dseg))
                return 0
            lax.fori_loop(0, DCH, dloop, 0)

        pltpu.async_copy(tok_b, out_hbm.at[b, pl.ds(p0 + o, CS)], sem_o)

    def wait_out(tok_b, sem_o):
        pltpu.make_async_copy(tok_b, out_hbm.at[0, pl.ds(0, CS)],
                              sem_o).wait()

    toks = (tok0, tok1, tok2, tok3)
    sgs = (sg0, sg1, sg2, sg3)
    sos = (so0, so1, so2, so3)

    def rowstep(r, _):
        compute(r, 0, toks[0], sgs[0], sos[0])

        @pl.when(r > 0)
        def _():
            wait_out(toks[3], sos[3])
            prefetch(r, 3 * CS, toks[3], sgs[3])

        for q in range(1, NSLOT):
            compute(r, q * CS, toks[q], sgs[q], sos[q])

            @pl.when(r < RPW - 1)
            def _():
                wait_out(toks[q - 1], sos[q - 1])
                prefetch(r + 1, (q - 1) * CS, toks[q - 1], sgs[q - 1])

        return 0

    lax.fori_loop(0, RPW, rowstep, 0)
    for q in range(NSLOT):
        wait_out(toks[q], sos[q])


@jax.jit
def kernel(token_ids, token_matrix, segment_matrix, pos_matrix):
    mesh = plsc.VectorSubcoreMesh(core_axis_name="c", subcore_axis_name="s",
                                  num_cores=NC, num_subcores=NS)
    run = pl.kernel(
        _body,
        out_type=jax.ShapeDtypeStruct((B, SEQ, D), jnp.float32),
        mesh=mesh,
        scratch_types=[
            pltpu.VMEM((RPW, SEQ), jnp.int32),
            pltpu.VMEM((RPW, PW), jnp.int32),
            pltpu.VMEM((RPW, PW), jnp.float32),
            pltpu.VMEM((2, D), jnp.float32),
            pltpu.VMEM((PW, D), jnp.float32),
            pltpu.VMEM((CS, D), jnp.float32),
            pltpu.VMEM((CS, D), jnp.float32),
            pltpu.VMEM((CS, D), jnp.float32),
            pltpu.VMEM((CS, D), jnp.float32),
            pltpu.VMEM((L,), jnp.int32),
            pltpu.SemaphoreType.DMA,
            pltpu.SemaphoreType.DMA,
            pltpu.SemaphoreType.DMA,
            pltpu.SemaphoreType.DMA,
            pltpu.SemaphoreType.DMA,
            pltpu.SemaphoreType.DMA,
            pltpu.SemaphoreType.DMA,
            pltpu.SemaphoreType.DMA,
            pltpu.SemaphoreType.DMA,
        ],
        compiler_params=pltpu.CompilerParams(needs_layout_passes=False),
    )
    return run(token_ids.astype(jnp.int32), token_matrix, segment_matrix,
               pos_matrix)
